# Initial kernel scaffold; baseline (speedup 1.0000x reference)
import jax
import jax.numpy as jnp
from jax.experimental import pallas as pl


def kernel(hidden_states, Wr, br, W1, b1, W2, b2):
    raise NotImplementedError("write your pallas kernel here")


# speedup vs baseline: 3.4798x; 3.4798x over previous
import functools

import jax
import jax.numpy as jnp
from jax.experimental import pallas as pl
from jax.experimental.pallas import tpu as pltpu

D_MODEL_ = 768
D_FF_ = 3072
N_EXP_ = 8
FT_ = 512
N_FT_ = D_FF_ // FT_


def _moe_body(x_ref, wr_ref, br_ref, w1_ref, b1_ref, w2_ref, b2_ref,
              out_ref, wnorm_ref, xb_ref):
    e = pl.program_id(0)
    f = pl.program_id(1)
    first = (e == 0) & (f == 0)

    @pl.when(first)
    def _router():
        x = x_ref[...]
        logits = jax.lax.dot_general(
            x, wr_ref[...], (((1,), (1,)), ((), ())),
            preferred_element_type=jnp.float32) + br_ref[...]
        m = jnp.max(logits, axis=-1, keepdims=True)
        ex = jnp.exp(logits - m)
        p = ex / jnp.sum(ex, axis=-1, keepdims=True)
        eidx = jax.lax.broadcasted_iota(jnp.int32, p.shape, 1)
        m1 = jnp.max(p, axis=-1, keepdims=True)
        i1 = jnp.min(jnp.where(p == m1, eidx, N_EXP_), axis=-1, keepdims=True)
        p2 = jnp.where(eidx == i1, -1.0, p)
        m2 = jnp.max(p2, axis=-1, keepdims=True)
        i2 = jnp.min(jnp.where(p2 == m2, eidx, N_EXP_), axis=-1, keepdims=True)
        denom = m1 + m2 + 1e-8
        wnorm = jnp.where(eidx == i1, m1 / denom,
                          jnp.where(eidx == i2, m2 / denom, 0.0))
        wnorm_ref[...] = wnorm
        xb_ref[...] = x.astype(jnp.bfloat16)
        out_ref[...] = x

    onehot = (jax.lax.broadcasted_iota(jnp.int32, (N_EXP_, 1), 0) == e
              ).astype(jnp.float32)
    w_col = jax.lax.dot_general(
        wnorm_ref[...], onehot, (((1,), (0,)), ((), ())),
        preferred_element_type=jnp.float32)

    xb = xb_ref[...]
    w1b = w1_ref[0].astype(jnp.bfloat16)
    h = jax.lax.dot_general(
        xb, w1b, (((1,), (1,)), ((), ())),
        preferred_element_type=jnp.float32)
    h = h + b1_ref[0, 0][None, :]
    h = 0.5 * h * (1.0 + jax.lax.erf(h * 0.7071067811865476))
    hb = h.astype(jnp.bfloat16)
    w2b = w2_ref[0].astype(jnp.bfloat16)
    y = jax.lax.dot_general(
        hb, w2b, (((1,), (1,)), ((), ())),
        preferred_element_type=jnp.float32)
    y = jnp.where(f == 0, y + b2_ref[0, 0][None, :], y)
    out_ref[...] += w_col * y


@functools.partial(jax.jit, static_argnames=())
def kernel(hidden_states, Wr, br, W1, b1, W2, b2):
    B, S, D = hidden_states.shape
    x = hidden_states.reshape(B * S, D)
    Sn = B * S
    br2 = br.reshape(1, N_EXP_)
    b1r = b1.reshape(N_EXP_, 1, D_FF_)
    b2r = b2.reshape(N_EXP_, 1, D_MODEL_)

    grid = (N_EXP_, N_FT_)
    out = pl.pallas_call(
        _moe_body,
        grid=grid,
        in_specs=[
            pl.BlockSpec((Sn, D), lambda e, f: (0, 0)),
            pl.BlockSpec((N_EXP_, D), lambda e, f: (0, 0)),
            pl.BlockSpec((1, N_EXP_), lambda e, f: (0, 0)),
            pl.BlockSpec((1, FT_, D), lambda e, f: (e, f, 0)),
            pl.BlockSpec((1, 1, FT_), lambda e, f: (e, 0, f)),
            pl.BlockSpec((1, D, FT_), lambda e, f: (e, 0, f)),
            pl.BlockSpec((1, 1, D), lambda e, f: (e, 0, 0)),
        ],
        out_specs=pl.BlockSpec((Sn, D), lambda e, f: (0, 0)),
        out_shape=jax.ShapeDtypeStruct((Sn, D), jnp.float32),
        scratch_shapes=[
            pltpu.VMEM((Sn, N_EXP_), jnp.float32),
            pltpu.VMEM((Sn, D), jnp.bfloat16),
        ],
    )(x, Wr, br2, W1, b1r, W2, b2r)
    return out.reshape(B, S, D)
